# baseline (speedup 1.0000x reference)
import functools

import jax
import jax.numpy as jnp
from jax import lax
from jax.experimental import pallas as pl
from jax.experimental.pallas import tpu as pltpu
from jax.experimental.pallas import tpu_sc as plsc

B = 256
S = 512
H = 768
HB = H // 16
EPS = 1e-12

NC = 2
NS = 16
NW = NC * NS
SBLK = S // NW
NB = 2
CHUNK = NB * SBLK
NCHUNK = B // NB
NBUF = 3


def _body(ids_hbm, tt_hbm, pid_hbm, wemb_hbm, pemb_hbm, temb_hbm, g_hbm, b_hbm,
          out_hbm,
          ids_v, tt_v, pid_v, rows, pos_v, type_v, diff_v,
          gamma_v, beta_v, idx_ring, ttf_ring, stats_m, stats_y, gsem, osem):
    wid = lax.axis_index("s") * NC + lax.axis_index("c")
    s0 = wid * SBLK

    lane = lax.iota(jnp.int32, 16)
    dnums = lax.GatherDimensionNumbers(
        offset_dims=(), collapsed_slice_dims=(0,), start_index_map=(0,))

    def vgather(v, idx):
        return lax.gather(v, idx[:, None], dnums, (1,),
                          mode=lax.GatherScatterMode.PROMISE_IN_BOUNDS)

    def allsum(v):
        for sh in (1, 2, 4, 8):
            v = v + vgather(v, lane ^ sh)
        return v

    pltpu.sync_copy(g_hbm, gamma_v)
    pltpu.sync_copy(b_hbm, beta_v)
    pltpu.sync_copy(temb_hbm, type_v)
    pltpu.sync_copy(ids_hbm.at[pl.ds(s0, SBLK), :], ids_v)
    pltpu.sync_copy(tt_hbm.at[pl.ds(s0, SBLK), :], tt_v)
    pltpu.sync_copy(pid_hbm.at[pl.ds(s0, SBLK)], pid_v)
    pltpu.async_copy(pemb_hbm.at[pid_v], pos_v, gsem.at[0]).wait()

    def fold(h, _):
        hs = pl.ds(h * 16, 16)
        t0 = type_v[0, hs]
        diff_v[hs] = type_v[1, hs] - t0
        for sl in range(SBLK):
            pos_v[sl, hs] = pos_v[sl, hs] + t0
        return 0
    lax.fori_loop(0, HB, fold, 0)

    def build_idx(c, rb):
        b0 = c * NB
        for i in range(NB):
            col = jnp.full((16,), b0 + i, jnp.int32)
            wv = plsc.load_gather(ids_v, [lane, col])
            idx_ring[rb, pl.ds(i * 16, 16)] = wv
            tv = plsc.load_gather(tt_v, [lane, col])
            ttf_ring[rb, pl.ds(i * 16, 16)] = tv.astype(jnp.float32)

    def start_gather(rb):
        pltpu.async_copy(wemb_hbm.at[idx_ring.at[rb]],
                         rows.at[pl.ds(rb * CHUNK, CHUNK), :], gsem.at[rb])

    def wait_gather(rb):
        pltpu.make_async_copy(wemb_hbm.at[idx_ring.at[rb]],
                              rows.at[pl.ds(rb * CHUNK, CHUNK), :],
                              gsem.at[rb]).wait()

    def start_out(c, rb):
        b0 = c * NB
        for i in range(NB):
            pltpu.async_copy(rows.at[pl.ds(rb * CHUNK + i * 16, 16), :],
                             out_hbm.at[b0 + i, pl.ds(s0, SBLK), :],
                             osem.at[rb])

    def drain_out(rb):
        pltpu.make_async_copy(wemb_hbm.at[pl.ds(0, CHUNK), :],
                              rows.at[pl.ds(rb * CHUNK, CHUNK), :],
                              osem.at[rb]).wait()

    def compute(c, rb):
        base = rb * CHUNK

        @plsc.parallel_loop(0, CHUNK, unroll=1)
        def token_step(t):
            i = t // 16
            sl = t - i * 16
            r = base + t
            ttrow = ttf_ring[rb, pl.ds(i * 16, 16)]
            ttv = vgather(ttrow, jnp.full((16,), sl, jnp.int32))

            a1 = [jnp.zeros((16,), jnp.float32) for _ in range(4)]
            a2 = [jnp.zeros((16,), jnp.float32) for _ in range(4)]
            for h in range(HB):
                hs = pl.ds(h * 16, 16)
                e = rows[r, hs] + pos_v[sl, hs] + ttv * diff_v[hs]
                rows[r, hs] = e
                a1[h % 4] = a1[h % 4] + e
                a2[h % 4] = a2[h % 4] + e * e
            stats_m[t, :] = (a1[0] + a1[1]) + (a1[2] + a1[3])
            stats_y[t, :] = (a2[0] + a2[1]) + (a2[2] + a2[3])

        @plsc.parallel_loop(0, CHUNK, unroll=2)
        def stats_step(t):
            mean_v = allsum(stats_m[t, :]) * (1.0 / H)
            var_v = allsum(stats_y[t, :]) * (1.0 / H) - mean_v * mean_v
            xv = var_v + EPS
            yi = jnp.int32(0x5F3759DF) - (plsc.bitcast(xv, jnp.int32) >> 1)
            y = plsc.bitcast(yi, jnp.float32)
            y = y * (1.5 - 0.5 * xv * y * y)
            y = y * (1.5 - 0.5 * xv * y * y)
            stats_m[t, :] = mean_v
            stats_y[t, :] = y

        @plsc.parallel_loop(0, CHUNK, unroll=2)
        def norm_step(t):
            r = base + t
            mean_v = stats_m[t, :]
            y = stats_y[t, :]
            for h in range(HB):
                hs = pl.ds(h * 16, 16)
                e = rows[r, hs]
                rows[r, hs] = (e - mean_v) * y * gamma_v[hs] + beta_v[hs]

    for c in range(NBUF - 1):
        build_idx(c, c)
        start_gather(c)

    def chunk_step(c, _):
        rb = lax.rem(c, NBUF)
        rbn = lax.rem(c + NBUF - 1, NBUF)

        @pl.when(c < NCHUNK - (NBUF - 1))
        def _prefetch():
            build_idx(c + NBUF - 1, rbn)

            @pl.when(c >= 1)
            def _():
                drain_out(rbn)
            start_gather(rbn)

        wait_gather(rb)
        compute(c, rb)
        start_out(c, rb)
        return 0
    lax.fori_loop(0, NCHUNK, chunk_step, 0)

    for rb in range(NBUF):
        drain_out(rb)


@jax.jit
def _run(ids_t, tt_t, pid_flat, word_emb, pos_emb, type_emb,
         ln_gamma, ln_beta):
    mesh = plsc.VectorSubcoreMesh(core_axis_name="c", subcore_axis_name="s")
    k = functools.partial(
        pl.kernel, mesh=mesh,
        compiler_params=pltpu.CompilerParams(needs_layout_passes=False),
        out_type=jax.ShapeDtypeStruct((B, S, H), jnp.float32),
        scratch_types=[
            pltpu.VMEM((SBLK, B), jnp.int32),
            pltpu.VMEM((SBLK, B), jnp.int32),
            pltpu.VMEM((SBLK,), jnp.int32),
            pltpu.VMEM((NBUF * CHUNK, H), jnp.float32),
            pltpu.VMEM((SBLK, H), jnp.float32),
            pltpu.VMEM((2, H), jnp.float32),
            pltpu.VMEM((H,), jnp.float32),
            pltpu.VMEM((H,), jnp.float32),
            pltpu.VMEM((H,), jnp.float32),
            pltpu.VMEM((NBUF, CHUNK), jnp.int32),
            pltpu.VMEM((NBUF, CHUNK), jnp.float32),
            pltpu.VMEM((CHUNK, 16), jnp.float32),
            pltpu.VMEM((CHUNK, 16), jnp.float32),
            pltpu.SemaphoreType.DMA((NBUF,)),
            pltpu.SemaphoreType.DMA((NBUF,)),
        ],
    )(_body)
    return k(ids_t, tt_t, pid_flat, word_emb, pos_emb, type_emb,
             ln_gamma, ln_beta)


def kernel(input_ids, token_type_ids, position_ids, word_emb, pos_emb,
           type_emb, ln_gamma, ln_beta):
    return _run(input_ids.astype(jnp.int32).T,
                token_type_ids.astype(jnp.int32).T,
                position_ids.astype(jnp.int32).reshape(-1),
                word_emb, pos_emb, type_emb, ln_gamma, ln_beta)


# speedup vs baseline: 1.1275x; 1.1275x over previous
import functools

import jax
import jax.numpy as jnp
from jax import lax
from jax.experimental import pallas as pl
from jax.experimental.pallas import tpu as pltpu
from jax.experimental.pallas import tpu_sc as plsc

B = 256
S = 512
H = 768
HB = H // 16
EPS = 1e-12

NC = 2
NS = 16
NW = NC * NS
SBLK = S // NW
NB = 2
CHUNK = NB * SBLK
NCHUNK = B // NB
NBUF = 3


def _body(ids_hbm, tt_hbm, pid_hbm, wemb_hbm, pemb_hbm, temb_hbm, g_hbm, b_hbm,
          out_hbm,
          ids_v, tt_v, pid_v, rows, pos2d, type_v,
          gamma_v, beta_v, idx_ring, tsel_ring, stats_m, stats_y, gsem, osem):
    wid = lax.axis_index("s") * NC + lax.axis_index("c")
    s0 = wid * SBLK

    lane = lax.iota(jnp.int32, 16)
    dnums = lax.GatherDimensionNumbers(
        offset_dims=(), collapsed_slice_dims=(0,), start_index_map=(0,))

    def vgather(v, idx):
        return lax.gather(v, idx[:, None], dnums, (1,),
                          mode=lax.GatherScatterMode.PROMISE_IN_BOUNDS)

    def allsum(v):
        for sh in (1, 2, 4, 8):
            v = v + vgather(v, lane ^ sh)
        return v

    pltpu.sync_copy(g_hbm, gamma_v)
    pltpu.sync_copy(b_hbm, beta_v)
    pltpu.sync_copy(temb_hbm, type_v)
    pltpu.sync_copy(ids_hbm.at[pl.ds(s0, SBLK), :], ids_v)
    pltpu.sync_copy(tt_hbm.at[pl.ds(s0, SBLK), :], tt_v)
    pltpu.sync_copy(pid_hbm.at[pl.ds(s0, SBLK)], pid_v)
    pltpu.async_copy(pemb_hbm.at[pid_v], rows.at[pl.ds(0, SBLK), :],
                     gsem.at[0]).wait()

    def fold(h, _):
        hs = pl.ds(h * 16, 16)
        for ty in range(2):
            tv = type_v[ty, hs]
            for sl in range(SBLK):
                pos2d[ty * SBLK + sl, hs] = rows[sl, hs] + tv
        return 0
    lax.fori_loop(0, HB, fold, 0)

    def build_idx(c, rb):
        b0 = c * NB
        for i in range(NB):
            col = jnp.full((16,), b0 + i, jnp.int32)
            wv = plsc.load_gather(ids_v, [lane, col])
            idx_ring[rb, pl.ds(i * 16, 16)] = wv
            tv = plsc.load_gather(tt_v, [lane, col])
            tsel_ring[rb, pl.ds(i * 16, 16)] = tv

    def start_gather(rb):
        pltpu.async_copy(wemb_hbm.at[idx_ring.at[rb]],
                         rows.at[pl.ds(rb * CHUNK, CHUNK), :], gsem.at[rb])

    def wait_gather(rb):
        pltpu.make_async_copy(wemb_hbm.at[idx_ring.at[rb]],
                              rows.at[pl.ds(rb * CHUNK, CHUNK), :],
                              gsem.at[rb]).wait()

    def start_out(c, rb):
        b0 = c * NB
        for i in range(NB):
            pltpu.async_copy(rows.at[pl.ds(rb * CHUNK + i * 16, 16), :],
                             out_hbm.at[b0 + i, pl.ds(s0, SBLK), :],
                             osem.at[rb])

    def drain_out(rb):
        pltpu.make_async_copy(wemb_hbm.at[pl.ds(0, CHUNK), :],
                              rows.at[pl.ds(rb * CHUNK, CHUNK), :],
                              osem.at[rb]).wait()

    def compute(c, rb):
        base = rb * CHUNK

        @plsc.parallel_loop(0, CHUNK, unroll=2)
        def token_step(t):
            i = t // 16
            sl = t - i * 16
            r = base + t
            ttrow = tsel_ring[rb, pl.ds(i * 16, 16)]
            ttv = vgather(ttrow, jnp.full((16,), sl, jnp.int32))
            pidx = jnp.max(ttv) * SBLK + sl

            a1 = [jnp.zeros((16,), jnp.float32) for _ in range(4)]
            a2 = [jnp.zeros((16,), jnp.float32) for _ in range(4)]
            for h in range(HB):
                hs = pl.ds(h * 16, 16)
                e = rows[r, hs] + pos2d[pidx, hs]
                rows[r, hs] = e
                a1[h % 4] = a1[h % 4] + e
                a2[h % 4] = a2[h % 4] + e * e
            stats_m[t, :] = (a1[0] + a1[1]) + (a1[2] + a1[3])
            stats_y[t, :] = (a2[0] + a2[1]) + (a2[2] + a2[3])

        @plsc.parallel_loop(0, CHUNK, unroll=2)
        def stats_step(t):
            mean_v = allsum(stats_m[t, :]) * (1.0 / H)
            var_v = allsum(stats_y[t, :]) * (1.0 / H) - mean_v * mean_v
            xv = var_v + EPS
            yi = jnp.int32(0x5F3759DF) - (plsc.bitcast(xv, jnp.int32) >> 1)
            y = plsc.bitcast(yi, jnp.float32)
            y = y * (1.5 - 0.5 * xv * y * y)
            y = y * (1.5 - 0.5 * xv * y * y)
            stats_m[t, :] = mean_v
            stats_y[t, :] = y

        @plsc.parallel_loop(0, CHUNK, unroll=2)
        def norm_step(t):
            r = base + t
            mean_v = stats_m[t, :]
            y = stats_y[t, :]
            for h in range(HB):
                hs = pl.ds(h * 16, 16)
                e = rows[r, hs]
                rows[r, hs] = (e - mean_v) * y * gamma_v[hs] + beta_v[hs]

    for c in range(NBUF - 1):
        build_idx(c, c)
        start_gather(c)

    def chunk_step(c, _):
        rb = lax.rem(c, NBUF)
        rbn = lax.rem(c + NBUF - 1, NBUF)

        @pl.when(c < NCHUNK - (NBUF - 1))
        def _prefetch():
            build_idx(c + NBUF - 1, rbn)

            @pl.when(c >= 1)
            def _():
                drain_out(rbn)
            start_gather(rbn)

        wait_gather(rb)
        compute(c, rb)
        start_out(c, rb)
        return 0
    lax.fori_loop(0, NCHUNK, chunk_step, 0)

    for rb in range(NBUF):
        drain_out(rb)


@jax.jit
def _run(ids_t, tt_t, pid_flat, word_emb, pos_emb, type_emb,
         ln_gamma, ln_beta):
    mesh = plsc.VectorSubcoreMesh(core_axis_name="c", subcore_axis_name="s")
    k = functools.partial(
        pl.kernel, mesh=mesh,
        compiler_params=pltpu.CompilerParams(needs_layout_passes=False),
        out_type=jax.ShapeDtypeStruct((B, S, H), jnp.float32),
        scratch_types=[
            pltpu.VMEM((SBLK, B), jnp.int32),
            pltpu.VMEM((SBLK, B), jnp.int32),
            pltpu.VMEM((SBLK,), jnp.int32),
            pltpu.VMEM((NBUF * CHUNK, H), jnp.float32),
            pltpu.VMEM((2 * SBLK, H), jnp.float32),
            pltpu.VMEM((2, H), jnp.float32),
            pltpu.VMEM((H,), jnp.float32),
            pltpu.VMEM((H,), jnp.float32),
            pltpu.VMEM((NBUF, CHUNK), jnp.int32),
            pltpu.VMEM((NBUF, CHUNK), jnp.int32),
            pltpu.VMEM((CHUNK, 16), jnp.float32),
            pltpu.VMEM((CHUNK, 16), jnp.float32),
            pltpu.SemaphoreType.DMA((NBUF,)),
            pltpu.SemaphoreType.DMA((NBUF,)),
        ],
    )(_body)
    return k(ids_t, tt_t, pid_flat, word_emb, pos_emb, type_emb,
             ln_gamma, ln_beta)


def kernel(input_ids, token_type_ids, position_ids, word_emb, pos_emb,
           type_emb, ln_gamma, ln_beta):
    return _run(input_ids.astype(jnp.int32).T,
                token_type_ids.astype(jnp.int32).T,
                position_ids.astype(jnp.int32).reshape(-1),
                word_emb, pos_emb, type_emb, ln_gamma, ln_beta)


# speedup vs baseline: 1.4099x; 1.2504x over previous
import functools

import jax
import jax.numpy as jnp
from jax import lax
from jax.experimental import pallas as pl
from jax.experimental.pallas import tpu as pltpu
from jax.experimental.pallas import tpu_sc as plsc

B = 256
S = 512
H = 768
HB = H // 16
EPS = 1e-12

NC = 2
NS = 16
NW = NC * NS
SBLK = S // NW
NB = 2
CHUNK = NB * SBLK
NCHUNK = B // NB
NBUF = 3


def _body(ids_hbm, tt_hbm, pid_hbm, wemb_hbm, pemb_hbm, temb_hbm, g_hbm, b_hbm,
          out_hbm,
          ids_v, tt_v, pid_v, rows, pos_v, type_v, diff_v,
          gamma_v, beta_v, idx_ring, ttf_ring, stats_m, stats_y, gsem, osem):
    wid = lax.axis_index("s") * NC + lax.axis_index("c")
    s0 = wid * SBLK

    lane = lax.iota(jnp.int32, 16)
    dnums = lax.GatherDimensionNumbers(
        offset_dims=(), collapsed_slice_dims=(0,), start_index_map=(0,))

    def vgather(v, idx):
        return lax.gather(v, idx[:, None], dnums, (1,),
                          mode=lax.GatherScatterMode.PROMISE_IN_BOUNDS)

    def allsum(v):
        for sh in (1, 2, 4, 8):
            v = v + vgather(v, lane ^ sh)
        return v

    pltpu.sync_copy(g_hbm, gamma_v)
    pltpu.sync_copy(b_hbm, beta_v)
    pltpu.sync_copy(temb_hbm, type_v)
    pltpu.sync_copy(ids_hbm.at[pl.ds(s0, SBLK), :], ids_v)
    pltpu.sync_copy(tt_hbm.at[pl.ds(s0, SBLK), :], tt_v)
    pltpu.sync_copy(pid_hbm.at[pl.ds(s0, SBLK)], pid_v)
    pltpu.async_copy(pemb_hbm.at[pid_v], pos_v, gsem.at[0]).wait()

    def fold(h, _):
        hs = pl.ds(h * 16, 16)
        t0 = type_v[0, hs]
        diff_v[hs] = type_v[1, hs] - t0
        for sl in range(SBLK):
            pos_v[sl, hs] = pos_v[sl, hs] + t0
        return 0
    lax.fori_loop(0, HB, fold, 0)

    def build_idx(c, rb):
        b0 = c * NB
        for i in range(NB):
            col = jnp.full((16,), b0 + i, jnp.int32)
            wv = plsc.load_gather(ids_v, [lane, col])
            idx_ring[rb, pl.ds(i * 16, 16)] = wv
            tv = plsc.load_gather(tt_v, [lane, col])
            ttf_ring[rb, pl.ds(i * 16, 16)] = tv.astype(jnp.float32)

    def start_gather(rb):
        pltpu.async_copy(wemb_hbm.at[idx_ring.at[rb]],
                         rows.at[pl.ds(rb * CHUNK, CHUNK), :], gsem.at[rb])

    def wait_gather(rb):
        pltpu.make_async_copy(wemb_hbm.at[idx_ring.at[rb]],
                              rows.at[pl.ds(rb * CHUNK, CHUNK), :],
                              gsem.at[rb]).wait()

    def start_out(c, rb):
        b0 = c * NB
        for i in range(NB):
            pltpu.async_copy(rows.at[pl.ds(rb * CHUNK + i * 16, 16), :],
                             out_hbm.at[b0 + i, pl.ds(s0, SBLK), :],
                             osem.at[rb])

    def drain_out(rb):
        pltpu.make_async_copy(wemb_hbm.at[pl.ds(0, CHUNK), :],
                              rows.at[pl.ds(rb * CHUNK, CHUNK), :],
                              osem.at[rb]).wait()

    def compute(c, rb):
        base = rb * CHUNK

        @plsc.parallel_loop(0, CHUNK, unroll=2)
        def token_step(t):
            i = t // 16
            sl = t - i * 16
            r = base + t
            ttrow = ttf_ring[rb, pl.ds(i * 16, 16)]
            ttv = vgather(ttrow, jnp.full((16,), sl, jnp.int32))

            a1 = [jnp.zeros((16,), jnp.float32) for _ in range(4)]
            a2 = [jnp.zeros((16,), jnp.float32) for _ in range(4)]
            for h in range(HB):
                hs = pl.ds(h * 16, 16)
                e = rows[r, hs] + pos_v[sl, hs] + ttv * diff_v[hs]
                rows[r, hs] = e
                a1[h % 4] = a1[h % 4] + e
                a2[h % 4] = a2[h % 4] + e * e
            stats_m[t, :] = (a1[0] + a1[1]) + (a1[2] + a1[3])
            stats_y[t, :] = (a2[0] + a2[1]) + (a2[2] + a2[3])

        @plsc.parallel_loop(0, CHUNK, unroll=2)
        def stats_step(t):
            mean_v = allsum(stats_m[t, :]) * (1.0 / H)
            var_v = allsum(stats_y[t, :]) * (1.0 / H) - mean_v * mean_v
            xv = var_v + EPS
            yi = jnp.int32(0x5F3759DF) - (plsc.bitcast(xv, jnp.int32) >> 1)
            y = plsc.bitcast(yi, jnp.float32)
            y = y * (1.5 - 0.5 * xv * y * y)
            y = y * (1.5 - 0.5 * xv * y * y)
            stats_m[t, :] = mean_v
            stats_y[t, :] = y

        @plsc.parallel_loop(0, CHUNK, unroll=2)
        def norm_step(t):
            r = base + t
            mean_v = stats_m[t, :]
            y = stats_y[t, :]
            for h in range(HB):
                hs = pl.ds(h * 16, 16)
                e = rows[r, hs]
                rows[r, hs] = (e - mean_v) * y * gamma_v[hs] + beta_v[hs]

    for c in range(NBUF - 1):
        build_idx(c, c)
        start_gather(c)

    def chunk_step(c, _):
        rb = lax.rem(c, NBUF)
        rbn = lax.rem(c + NBUF - 1, NBUF)

        @pl.when(c < NCHUNK - (NBUF - 1))
        def _prefetch():
            build_idx(c + NBUF - 1, rbn)

            @pl.when(c >= 1)
            def _():
                drain_out(rbn)
            start_gather(rbn)

        wait_gather(rb)
        compute(c, rb)
        start_out(c, rb)
        return 0
    lax.fori_loop(0, NCHUNK, chunk_step, 0)

    for rb in range(NBUF):
        drain_out(rb)


@jax.jit
def _run(ids_t, tt_t, pid_flat, word_emb, pos_emb, type_emb,
         ln_gamma, ln_beta):
    mesh = plsc.VectorSubcoreMesh(core_axis_name="c", subcore_axis_name="s")
    k = functools.partial(
        pl.kernel, mesh=mesh,
        compiler_params=pltpu.CompilerParams(needs_layout_passes=False),
        out_type=jax.ShapeDtypeStruct((B, S, H), jnp.float32),
        scratch_types=[
            pltpu.VMEM((SBLK, B), jnp.int32),
            pltpu.VMEM((SBLK, B), jnp.int32),
            pltpu.VMEM((SBLK,), jnp.int32),
            pltpu.VMEM((NBUF * CHUNK, H), jnp.float32),
            pltpu.VMEM((SBLK, H), jnp.float32),
            pltpu.VMEM((2, H), jnp.float32),
            pltpu.VMEM((H,), jnp.float32),
            pltpu.VMEM((H,), jnp.float32),
            pltpu.VMEM((H,), jnp.float32),
            pltpu.VMEM((NBUF, CHUNK), jnp.int32),
            pltpu.VMEM((NBUF, CHUNK), jnp.float32),
            pltpu.VMEM((CHUNK, 16), jnp.float32),
            pltpu.VMEM((CHUNK, 16), jnp.float32),
            pltpu.SemaphoreType.DMA((NBUF,)),
            pltpu.SemaphoreType.DMA((NBUF,)),
        ],
    )(_body)
    return k(ids_t, tt_t, pid_flat, word_emb, pos_emb, type_emb,
             ln_gamma, ln_beta)


def kernel(input_ids, token_type_ids, position_ids, word_emb, pos_emb,
           type_emb, ln_gamma, ln_beta):
    return _run(input_ids.astype(jnp.int32).T,
                token_type_ids.astype(jnp.int32).T,
                position_ids.astype(jnp.int32).reshape(-1),
                word_emb, pos_emb, type_emb, ln_gamma, ln_beta)
